# initial kernel scaffold (unmeasured)
import jax
import jax.numpy as jnp
from jax import lax
from jax.experimental import pallas as pl
from jax.experimental.pallas import tpu as pltpu

SQ = 2048
D = 1024
HQ = 8
DH = 128
BLK = 64
SCALE = 0.08838834764831843
QT = 512
N_TILES = SQ // QT


def kernel(x, Wq, K_ext, V_ext, Wo):
    x2 = x.reshape(SQ, D)
    K2 = K_ext.reshape(SQ, D)
    V2 = V_ext.reshape(SQ, D)

    def body(x_hbm, wq_ref, k_hbm, v_hbm, wo_ref, out_ref,
             stage, xbf, kbf, vbf, qbf, ctx_tile,
             copy_sem, send_sems, recv_sem):
        my = lax.axis_index("i")

        @pl.when(my == 0)
        def _producer():
            for src, dst in ((x_hbm, xbf), (k_hbm, kbf), (v_hbm, vbf)):
                cp = pltpu.make_async_copy(src, stage, copy_sem)
                cp.start()
                cp.wait()
                dst[...] = stage[...].astype(jnp.bfloat16)

            q = jnp.dot(xbf[...], wq_ref[...].astype(jnp.bfloat16),
                        preferred_element_type=jnp.float32)
            qbf[...] = q.astype(jnp.bfloat16)
            wo_bf = wo_ref[...].astype(jnp.bfloat16)

            for t in range(N_TILES):
                r0 = t * QT
                kv = r0 + QT
                for h in range(HQ):
                    qt = qbf[r0:r0 + QT, h * DH:(h + 1) * DH]
                    kt = kbf[0:kv, h * DH:(h + 1) * DH]
                    s = lax.dot_general(
                        qt, kt, (((1,), (1,)), ((), ())),
                        preferred_element_type=jnp.float32) * SCALE
                    rows = r0 + lax.broadcasted_iota(jnp.int32, (QT, kv), 0)
                    cols = lax.broadcasted_iota(jnp.int32, (QT, kv), 1)
                    s = jnp.where((rows // BLK) >= (cols // BLK), s, -1e9)
                    m = jnp.max(s, axis=1, keepdims=True)
                    w = jnp.exp(s - m)
                    w = w / jnp.sum(w, axis=1, keepdims=True)
                    vt = vbf[0:kv, h * DH:(h + 1) * DH]
                    ctx = lax.dot_general(
                        w.astype(jnp.bfloat16), vt, (((1,), (0,)), ((), ())),
                        preferred_element_type=jnp.float32)
                    ctx_tile[:, h * DH:(h + 1) * DH] = ctx.astype(jnp.bfloat16)
                ot = jnp.dot(ctx_tile[...], wo_bf,
                             preferred_element_type=jnp.float32)
                out_ref[r0:r0 + QT, :] = ot.astype(jnp.bfloat16)

            rdmas = []
            for d in (1, 2, 3):
                rd = pltpu.make_async_remote_copy(
                    src_ref=out_ref, dst_ref=out_ref,
                    send_sem=send_sems.at[d - 1], recv_sem=recv_sem,
                    device_id=(d,), device_id_type=pl.DeviceIdType.MESH)
                rd.start()
                rdmas.append(rd)
            for rd in rdmas:
                rd.wait_send()

        @pl.when(my != 0)
        def _consumer():
            rd = pltpu.make_async_remote_copy(
                src_ref=out_ref, dst_ref=out_ref,
                send_sem=send_sems.at[0], recv_sem=recv_sem,
                device_id=(0,), device_id_type=pl.DeviceIdType.MESH)
            rd.wait_recv()

    out = pl.pallas_call(
        body,
        out_shape=jax.ShapeDtypeStruct((SQ, D), jnp.bfloat16),
        in_specs=[
            pl.BlockSpec(memory_space=pltpu.ANY),
            pl.BlockSpec(memory_space=pltpu.VMEM),
            pl.BlockSpec(memory_space=pltpu.ANY),
            pl.BlockSpec(memory_space=pltpu.ANY),
            pl.BlockSpec(memory_space=pltpu.VMEM),
        ],
        out_specs=pl.BlockSpec(memory_space=pltpu.VMEM),
        scratch_shapes=[
            pltpu.VMEM((SQ, D), jnp.float32),
            pltpu.VMEM((SQ, D), jnp.bfloat16),
            pltpu.VMEM((SQ, D), jnp.bfloat16),
            pltpu.VMEM((SQ, D), jnp.bfloat16),
            pltpu.VMEM((SQ, D), jnp.bfloat16),
            pltpu.VMEM((QT, D), jnp.bfloat16),
            pltpu.SemaphoreType.DMA,
            pltpu.SemaphoreType.DMA((3,)),
            pltpu.SemaphoreType.DMA,
        ],
    )(x2, Wq, K2, V2, Wo)

    return out.reshape(1, SQ, D)


# baseline (device time: 169225 ns/iter reference)
import jax
import jax.numpy as jnp
from jax import lax
from jax.experimental import pallas as pl
from jax.experimental.pallas import tpu as pltpu

SQ = 2048
D = 1024
HQ = 8
DH = 128
BLK = 64
SCALE = 0.08838834764831843
QT = 512
N_TILES = SQ // QT


def kernel(x, Wq, K_ext, V_ext, Wo):
    x2 = x.reshape(SQ, D)
    K2 = K_ext.reshape(SQ, D)
    V2 = V_ext.reshape(SQ, D)

    def body(x_hbm, wq_ref, k_hbm, v_hbm, wo_ref, out_ref,
             stage, xbf, kbf, vbf, qbf, ctx_tile,
             copy_sem, send_sems, recv_sem):
        my = lax.axis_index("i")

        @pl.when(my == 0)
        def _producer():
            for src, dst in ((x_hbm, xbf), (k_hbm, kbf), (v_hbm, vbf)):
                cp = pltpu.make_async_copy(src, stage, copy_sem)
                cp.start()
                cp.wait()
                dst[...] = stage[...].astype(jnp.bfloat16)

            q = jnp.dot(xbf[...], wq_ref[...].astype(jnp.bfloat16),
                        preferred_element_type=jnp.float32)
            qbf[...] = q.astype(jnp.bfloat16)
            wo_bf = wo_ref[...].astype(jnp.bfloat16)

            for t in range(N_TILES):
                r0 = t * QT
                kv = r0 + QT
                for h in range(HQ):
                    qt = qbf[r0:r0 + QT, h * DH:(h + 1) * DH]
                    kt = kbf[0:kv, h * DH:(h + 1) * DH]
                    s = lax.dot_general(
                        qt, kt, (((1,), (1,)), ((), ())),
                        preferred_element_type=jnp.float32) * SCALE
                    rows = r0 + lax.broadcasted_iota(jnp.int32, (QT, kv), 0)
                    cols = lax.broadcasted_iota(jnp.int32, (QT, kv), 1)
                    s = jnp.where((rows // BLK) >= (cols // BLK), s, -1e9)
                    m = jnp.max(s, axis=1, keepdims=True)
                    w = jnp.exp(s - m)
                    w = w / jnp.sum(w, axis=1, keepdims=True)
                    vt = vbf[0:kv, h * DH:(h + 1) * DH]
                    ctx = lax.dot_general(
                        w.astype(jnp.bfloat16), vt, (((1,), (0,)), ((), ())),
                        preferred_element_type=jnp.float32)
                    ctx_tile[:, h * DH:(h + 1) * DH] = ctx.astype(jnp.bfloat16)
                ot = jnp.dot(ctx_tile[...], wo_bf,
                             preferred_element_type=jnp.float32)
                out_ref[r0:r0 + QT, :] = ot.astype(jnp.bfloat16)

            rdmas = []
            for d in (1, 2, 3):
                rd = pltpu.make_async_remote_copy(
                    src_ref=out_ref, dst_ref=out_ref,
                    send_sem=send_sems.at[d - 1], recv_sem=recv_sem,
                    device_id=(d,), device_id_type=pl.DeviceIdType.MESH)
                rd.start()
                rdmas.append(rd)
            for rd in rdmas:
                rd.wait_send()

        @pl.when(my != 0)
        def _consumer():
            rd = pltpu.make_async_remote_copy(
                src_ref=out_ref, dst_ref=out_ref,
                send_sem=send_sems.at[0], recv_sem=recv_sem,
                device_id=(0,), device_id_type=pl.DeviceIdType.MESH)
            rd.wait_recv()

    out = pl.pallas_call(
        body,
        out_shape=jax.ShapeDtypeStruct((SQ, D), jnp.bfloat16),
        in_specs=[
            pl.BlockSpec(memory_space=pl.ANY),
            pl.BlockSpec(memory_space=pltpu.VMEM),
            pl.BlockSpec(memory_space=pl.ANY),
            pl.BlockSpec(memory_space=pl.ANY),
            pl.BlockSpec(memory_space=pltpu.VMEM),
        ],
        out_specs=pl.BlockSpec(memory_space=pltpu.VMEM),
        scratch_shapes=[
            pltpu.VMEM((SQ, D), jnp.float32),
            pltpu.VMEM((SQ, D), jnp.bfloat16),
            pltpu.VMEM((SQ, D), jnp.bfloat16),
            pltpu.VMEM((SQ, D), jnp.bfloat16),
            pltpu.VMEM((SQ, D), jnp.bfloat16),
            pltpu.VMEM((QT, D), jnp.bfloat16),
            pltpu.SemaphoreType.DMA,
            pltpu.SemaphoreType.DMA((3,)),
            pltpu.SemaphoreType.DMA,
        ],
    )(x2, Wq, K2, V2, Wo)

    return out.reshape(1, SQ, D)


# device time: 121149 ns/iter; 1.3968x vs baseline; 1.3968x over previous
import jax
import jax.numpy as jnp
from jax import lax
from jax.experimental import pallas as pl
from jax.experimental.pallas import tpu as pltpu

SQ = 2048
D = 1024
HQ = 8
DH = 128
BLK = 64
SCALE = 0.08838834764831843
QT = 512
N_TILES = SQ // QT


def kernel(x, Wq, K_ext, V_ext, Wo):
    bf = jnp.bfloat16
    x2 = x.reshape(SQ, D).astype(bf)
    K2 = K_ext.reshape(SQ, D).astype(bf)
    V2 = V_ext.reshape(SQ, D).astype(bf)
    Wqb = Wq.astype(bf)
    Wob = Wo.astype(bf)

    def body(x_ref, wq_ref, k_ref, v_ref, wo_ref, out_ref,
             qbf, ctx_tile, send_sems, recv_sems):
        my = lax.axis_index("i")

        @pl.when(my == 0)
        def _producer():
            qbf[...] = jnp.dot(x_ref[...], wq_ref[...],
                               preferred_element_type=jnp.float32).astype(bf)
            rdmas = []
            for t in range(N_TILES):
                r0 = t * QT
                kv = r0 + QT
                rows = r0 + lax.broadcasted_iota(jnp.int32, (QT, kv), 0)
                cols = lax.broadcasted_iota(jnp.int32, (QT, kv), 1)
                bias = jnp.where((rows // BLK) >= (cols // BLK),
                                 jnp.float32(0), jnp.float32(-1e9))
                for h in range(HQ):
                    qt = qbf[r0:r0 + QT, h * DH:(h + 1) * DH]
                    kt = k_ref[0:kv, h * DH:(h + 1) * DH]
                    s = lax.dot_general(
                        qt, kt, (((1,), (1,)), ((), ())),
                        preferred_element_type=jnp.float32) * SCALE + bias
                    w = jnp.exp(s)
                    denom = jnp.sum(w, axis=1, keepdims=True)
                    vt = v_ref[0:kv, h * DH:(h + 1) * DH]
                    ctx = lax.dot_general(
                        w.astype(bf), vt, (((1,), (0,)), ((), ())),
                        preferred_element_type=jnp.float32) / denom
                    ctx_tile[:, h * DH:(h + 1) * DH] = ctx.astype(bf)
                ot = jnp.dot(ctx_tile[...], wo_ref[...],
                             preferred_element_type=jnp.float32)
                out_ref[r0:r0 + QT, :] = ot.astype(bf)
                for d in (1, 2, 3):
                    rd = pltpu.make_async_remote_copy(
                        src_ref=out_ref.at[r0:r0 + QT, :],
                        dst_ref=out_ref.at[r0:r0 + QT, :],
                        send_sem=send_sems.at[t, d - 1],
                        recv_sem=recv_sems.at[t],
                        device_id=(d,), device_id_type=pl.DeviceIdType.MESH)
                    rd.start()
                    rdmas.append(rd)
            for rd in rdmas:
                rd.wait_send()

        @pl.when(my != 0)
        def _consumer():
            for t in range(N_TILES):
                r0 = t * QT
                rd = pltpu.make_async_remote_copy(
                    src_ref=out_ref.at[r0:r0 + QT, :],
                    dst_ref=out_ref.at[r0:r0 + QT, :],
                    send_sem=send_sems.at[t, 0], recv_sem=recv_sems.at[t],
                    device_id=(0,), device_id_type=pl.DeviceIdType.MESH)
                rd.wait_recv()

    out = pl.pallas_call(
        body,
        out_shape=jax.ShapeDtypeStruct((SQ, D), bf),
        in_specs=[pl.BlockSpec(memory_space=pltpu.VMEM)] * 5,
        out_specs=pl.BlockSpec(memory_space=pltpu.VMEM),
        scratch_shapes=[
            pltpu.VMEM((SQ, D), bf),
            pltpu.VMEM((QT, D), bf),
            pltpu.SemaphoreType.DMA((N_TILES, 3)),
            pltpu.SemaphoreType.DMA((N_TILES,)),
        ],
    )(x2, Wqb, K2, V2, Wob)

    return out.reshape(1, SQ, D)


# device time: 87835 ns/iter; 1.9266x vs baseline; 1.3793x over previous
import jax
import jax.numpy as jnp
from jax import lax
from jax.experimental import pallas as pl
from jax.experimental.pallas import tpu as pltpu

SQ = 2048
D = 1024
HQ = 8
DH = 128
BLK = 64
SCALE = 0.08838834764831843
QT = 512
N_TILES = SQ // QT
NT = (((1,), (1,)), ((), ()))
NN = (((1,), (0,)), ((), ()))


def kernel(x, Wq, K_ext, V_ext, Wo):
    bf = jnp.bfloat16
    x2 = x.reshape(SQ, D).astype(bf)
    K2 = K_ext.reshape(SQ, D).astype(bf)
    V2 = V_ext.reshape(SQ, D).astype(bf)
    Wqb = Wq.astype(bf)
    Wob = Wo.astype(bf)

    def body(x_ref, wq_ref, k_ref, v_ref, wo_ref, out_ref,
             qbf, ctx_tile, send_sems, recv_sems, fwd_sems):
        my = lax.axis_index("i")

        @pl.when(my == 0)
        def _producer():
            qbf[...] = jnp.dot(x_ref[...], wq_ref[...],
                               preferred_element_type=jnp.float32).astype(bf)
            r = lax.broadcasted_iota(jnp.int32, (QT, QT), 0) // BLK
            c = lax.broadcasted_iota(jnp.int32, (QT, QT), 1) // BLK
            bias_diag = jnp.where(r >= c, jnp.float32(0), jnp.float32(-1e9))

            rdmas = []
            for t in range(N_TILES):
                r0 = t * QT
                for h in range(HQ):
                    hs = slice(h * DH, (h + 1) * DH)
                    qt = qbf[r0:r0 + QT, hs]
                    sd = lax.dot_general(
                        qt, k_ref[r0:r0 + QT, hs], NT,
                        preferred_element_type=jnp.float32) * SCALE + bias_diag
                    wd = jnp.exp(sd)
                    denom = jnp.sum(wd, axis=1, keepdims=True)
                    ctx = lax.dot_general(
                        wd.astype(bf), v_ref[r0:r0 + QT, hs], NN,
                        preferred_element_type=jnp.float32)
                    if t > 0:
                        s1 = lax.dot_general(
                            qt, k_ref[0:r0, hs], NT,
                            preferred_element_type=jnp.float32) * SCALE
                        w1 = jnp.exp(s1)
                        denom = denom + jnp.sum(w1, axis=1, keepdims=True)
                        ctx = ctx + lax.dot_general(
                            w1.astype(bf), v_ref[0:r0, hs], NN,
                            preferred_element_type=jnp.float32)
                    ctx_tile[:, hs] = (ctx / denom).astype(bf)
                out_ref[r0:r0 + QT, :] = jnp.dot(
                    ctx_tile[...], wo_ref[...],
                    preferred_element_type=jnp.float32).astype(bf)
                for i, d in enumerate((1, 3)):
                    rd = pltpu.make_async_remote_copy(
                        src_ref=out_ref.at[r0:r0 + QT, :],
                        dst_ref=out_ref.at[r0:r0 + QT, :],
                        send_sem=send_sems.at[t, i], recv_sem=recv_sems.at[t],
                        device_id=(d,), device_id_type=pl.DeviceIdType.MESH)
                    rd.start()
                    rdmas.append(rd)
            for rd in rdmas:
                rd.wait_send()

        @pl.when(my != 0)
        def _consumer():
            fwd_conds = []
            for t in range(N_TILES):
                r0 = t * QT
                tile_src = out_ref.at[r0:r0 + QT, :]
                rd = pltpu.make_async_remote_copy(
                    src_ref=tile_src, dst_ref=tile_src,
                    send_sem=send_sems.at[t, 0], recv_sem=recv_sems.at[t],
                    device_id=(0,), device_id_type=pl.DeviceIdType.MESH)
                rd.wait_recv()
                cond = (my == 1) if t < 2 else (my == 3)
                fwd_conds.append(cond)

                @pl.when(cond)
                def _fwd(tile_src=tile_src, t=t):
                    f = pltpu.make_async_remote_copy(
                        src_ref=tile_src, dst_ref=tile_src,
                        send_sem=fwd_sems.at[t], recv_sem=recv_sems.at[t],
                        device_id=(2,), device_id_type=pl.DeviceIdType.MESH)
                    f.start()

            for t in range(N_TILES):
                r0 = t * QT
                tile_src = out_ref.at[r0:r0 + QT, :]

                @pl.when(fwd_conds[t])
                def _fwd_wait(tile_src=tile_src, t=t):
                    f = pltpu.make_async_remote_copy(
                        src_ref=tile_src, dst_ref=tile_src,
                        send_sem=fwd_sems.at[t], recv_sem=recv_sems.at[t],
                        device_id=(2,), device_id_type=pl.DeviceIdType.MESH)
                    f.wait_send()

    out = pl.pallas_call(
        body,
        out_shape=jax.ShapeDtypeStruct((SQ, D), bf),
        in_specs=[pl.BlockSpec(memory_space=pltpu.VMEM)] * 5,
        out_specs=pl.BlockSpec(memory_space=pltpu.VMEM),
        scratch_shapes=[
            pltpu.VMEM((SQ, D), bf),
            pltpu.VMEM((QT, D), bf),
            pltpu.SemaphoreType.DMA((N_TILES, 2)),
            pltpu.SemaphoreType.DMA((N_TILES,)),
            pltpu.SemaphoreType.DMA((N_TILES,)),
        ],
    )(x2, Wqb, K2, V2, Wob)

    return out.reshape(1, SQ, D)


# device time: 81895 ns/iter; 2.0664x vs baseline; 1.0725x over previous
import jax
import jax.numpy as jnp
from jax import lax
from jax.experimental import pallas as pl
from jax.experimental.pallas import tpu as pltpu

SQ = 2048
D = 1024
HQ = 8
DH = 128
BLK = 64
SCALE = 0.08838834764831843
QT = 512
N_TILES = SQ // QT
CH = 256
N_CH = SQ // CH
NT = (((1,), (1,)), ((), ()))
NN = (((1,), (0,)), ((), ()))


def kernel(x, Wq, K_ext, V_ext, Wo):
    bf = jnp.bfloat16
    f32 = jnp.float32
    x2 = x.reshape(SQ, D).astype(bf)
    K2 = K_ext.reshape(SQ, D).astype(bf)
    V2 = V_ext.reshape(SQ, D).astype(bf)
    Wqb = Wq.astype(bf)
    Wob = Wo.astype(bf)

    def body(x_ref, wq_ref, k_ref, v_ref, wo_ref, out_ref,
             qbf, ctx_tile, send_sems, recv_sems, fwd_sems):
        my = lax.axis_index("i")

        def chunk_ref(c):
            return out_ref.at[c * CH:(c + 1) * CH, :]

        @pl.when(my == 0)
        def _producer():
            qbf[...] = jnp.dot(x_ref[...], wq_ref[...],
                               preferred_element_type=f32).astype(bf)
            r = lax.broadcasted_iota(jnp.int32, (QT, QT), 0) // BLK
            c = lax.broadcasted_iota(jnp.int32, (QT, QT), 1) // BLK
            bias_diag = jnp.where(r >= c, f32(0), f32(-1e9)).astype(bf)

            rdmas = []
            for t in range(N_TILES):
                r0 = t * QT
                for h in range(HQ):
                    hs = slice(h * DH, (h + 1) * DH)
                    qt = qbf[r0:r0 + QT, hs]
                    sd = lax.dot_general(
                        qt, k_ref[r0:r0 + QT, hs], NT,
                        preferred_element_type=f32)
                    wd = jnp.exp((sd * SCALE).astype(bf) + bias_diag)
                    denom = jnp.sum(wd, axis=1, keepdims=True, dtype=f32)
                    ctx = lax.dot_general(
                        wd, v_ref[r0:r0 + QT, hs], NN,
                        preferred_element_type=f32)
                    if t > 0:
                        s1 = lax.dot_general(
                            qt, k_ref[0:r0, hs], NT,
                            preferred_element_type=f32)
                        w1 = jnp.exp((s1 * SCALE).astype(bf))
                        denom = denom + jnp.sum(w1, axis=1, keepdims=True,
                                                dtype=f32)
                        ctx = ctx + lax.dot_general(
                            w1, v_ref[0:r0, hs], NN,
                            preferred_element_type=f32)
                    ctx_tile[:, hs] = (ctx / denom).astype(bf)
                out_ref[r0:r0 + QT, :] = jnp.dot(
                    ctx_tile[...], wo_ref[...],
                    preferred_element_type=f32).astype(bf)
                for cc in (2 * t, 2 * t + 1):
                    for i, d in enumerate((1, 3)):
                        rd = pltpu.make_async_remote_copy(
                            src_ref=chunk_ref(cc), dst_ref=chunk_ref(cc),
                            send_sem=send_sems.at[cc, i],
                            recv_sem=recv_sems.at[cc],
                            device_id=(d,),
                            device_id_type=pl.DeviceIdType.MESH)
                        rd.start()
                        rdmas.append(rd)
            for rd in rdmas:
                rd.wait_send()

        @pl.when(my != 0)
        def _consumer():
            fwd_conds = []
            for cc in range(N_CH):
                rd = pltpu.make_async_remote_copy(
                    src_ref=chunk_ref(cc), dst_ref=chunk_ref(cc),
                    send_sem=send_sems.at[cc, 0], recv_sem=recv_sems.at[cc],
                    device_id=(0,), device_id_type=pl.DeviceIdType.MESH)
                rd.wait_recv()
                cond = (my == 1) if cc < N_CH // 2 else (my == 3)
                fwd_conds.append(cond)

                @pl.when(cond)
                def _fwd(cc=cc):
                    f = pltpu.make_async_remote_copy(
                        src_ref=chunk_ref(cc), dst_ref=chunk_ref(cc),
                        send_sem=fwd_sems.at[cc], recv_sem=recv_sems.at[cc],
                        device_id=(2,), device_id_type=pl.DeviceIdType.MESH)
                    f.start()

            for cc in range(N_CH):
                @pl.when(fwd_conds[cc])
                def _fwd_wait(cc=cc):
                    f = pltpu.make_async_remote_copy(
                        src_ref=chunk_ref(cc), dst_ref=chunk_ref(cc),
                        send_sem=fwd_sems.at[cc], recv_sem=recv_sems.at[cc],
                        device_id=(2,), device_id_type=pl.DeviceIdType.MESH)
                    f.wait_send()

    out = pl.pallas_call(
        body,
        out_shape=jax.ShapeDtypeStruct((SQ, D), bf),
        in_specs=[pl.BlockSpec(memory_space=pltpu.VMEM)] * 5,
        out_specs=pl.BlockSpec(memory_space=pltpu.VMEM),
        scratch_shapes=[
            pltpu.VMEM((SQ, D), bf),
            pltpu.VMEM((QT, D), bf),
            pltpu.SemaphoreType.DMA((N_CH, 2)),
            pltpu.SemaphoreType.DMA((N_CH,)),
            pltpu.SemaphoreType.DMA((N_CH,)),
        ],
    )(x2, Wqb, K2, V2, Wob)

    return out.reshape(1, SQ, D)


# device time: 77194 ns/iter; 2.1922x vs baseline; 1.0609x over previous
import jax
import jax.numpy as jnp
from jax import lax
from jax.experimental import pallas as pl
from jax.experimental.pallas import tpu as pltpu

SQ = 2048
D = 1024
HQ = 8
DH = 128
BLK = 64
SCALE = 0.08838834764831843
QT = 256
N_TILES = SQ // QT
NT = (((1,), (1,)), ((), ()))
NN = (((1,), (0,)), ((), ()))


def kernel(x, Wq, K_ext, V_ext, Wo):
    bf = jnp.bfloat16
    f32 = jnp.float32
    x2 = x.reshape(SQ, D).astype(bf)
    K2 = K_ext.reshape(SQ, D).astype(bf)
    V2 = V_ext.reshape(SQ, D).astype(bf)
    Wqb = Wq.astype(bf)
    Wob = Wo.astype(bf)

    def body(x_ref, wq_ref, k_ref, v_ref, wo_ref, out_ref,
             qbf, ctx_tile, send_sems, recv_sems, fwd_sems):
        my = lax.axis_index("i")

        def tile_ref(t):
            return out_ref.at[t * QT:(t + 1) * QT, :]

        barrier = pltpu.get_barrier_semaphore()

        @pl.when((my == 1) | (my == 3))
        def _():
            pl.semaphore_signal(barrier, inc=1, device_id=(0,),
                                device_id_type=pl.DeviceIdType.MESH)
            pl.semaphore_wait(barrier, 1)

        @pl.when(my == 2)
        def _():
            for d in (1, 3):
                pl.semaphore_signal(barrier, inc=1, device_id=(d,),
                                    device_id_type=pl.DeviceIdType.MESH)

        @pl.when(my == 0)
        def _():
            pl.semaphore_wait(barrier, 2)

        @pl.when(my == 0)
        def _producer():
            qbf[...] = jnp.dot(x_ref[...], wq_ref[...],
                               preferred_element_type=f32).astype(bf)
            r = lax.broadcasted_iota(jnp.int32, (QT, QT), 0) // BLK
            c = lax.broadcasted_iota(jnp.int32, (QT, QT), 1) // BLK
            bias_diag = jnp.where(r >= c, f32(0), f32(-1e9)).astype(bf)

            rdmas = []
            for t in range(N_TILES):
                r0 = t * QT
                for h in range(HQ):
                    hs = slice(h * DH, (h + 1) * DH)
                    qt = qbf[r0:r0 + QT, hs]
                    sd = lax.dot_general(
                        qt, k_ref[r0:r0 + QT, hs], NT,
                        preferred_element_type=f32)
                    wd = jnp.exp((sd * SCALE).astype(bf) + bias_diag)
                    denom = jnp.sum(wd, axis=1, keepdims=True, dtype=f32)
                    ctx = lax.dot_general(
                        wd, v_ref[r0:r0 + QT, hs], NN,
                        preferred_element_type=f32)
                    if t > 0:
                        s1 = lax.dot_general(
                            qt, k_ref[0:r0, hs], NT,
                            preferred_element_type=f32)
                        w1 = jnp.exp((s1 * SCALE).astype(bf))
                        denom = denom + jnp.sum(w1, axis=1, keepdims=True,
                                                dtype=f32)
                        ctx = ctx + lax.dot_general(
                            w1, v_ref[0:r0, hs], NN,
                            preferred_element_type=f32)
                    ctx_tile[:, hs] = (ctx / denom).astype(bf)
                out_ref[r0:r0 + QT, :] = jnp.dot(
                    ctx_tile[...], wo_ref[...],
                    preferred_element_type=f32).astype(bf)
                for i, d in enumerate((1, 3)):
                    rd = pltpu.make_async_remote_copy(
                        src_ref=tile_ref(t), dst_ref=tile_ref(t),
                        send_sem=send_sems.at[t, i], recv_sem=recv_sems.at[t],
                        device_id=(d,), device_id_type=pl.DeviceIdType.MESH)
                    rd.start()
                    rdmas.append(rd)
            for rd in rdmas:
                rd.wait_send()

        @pl.when(my != 0)
        def _consumer():
            fwd_conds = []
            for t in range(N_TILES):
                rd = pltpu.make_async_remote_copy(
                    src_ref=tile_ref(t), dst_ref=tile_ref(t),
                    send_sem=send_sems.at[t, 0], recv_sem=recv_sems.at[t],
                    device_id=(0,), device_id_type=pl.DeviceIdType.MESH)
                rd.wait_recv()
                cond = (my == 1) if t % 2 == 0 else (my == 3)
                fwd_conds.append(cond)

                @pl.when(cond)
                def _fwd(t=t):
                    f = pltpu.make_async_remote_copy(
                        src_ref=tile_ref(t), dst_ref=tile_ref(t),
                        send_sem=fwd_sems.at[t], recv_sem=recv_sems.at[t],
                        device_id=(2,), device_id_type=pl.DeviceIdType.MESH)
                    f.start()

            for t in range(N_TILES):
                @pl.when(fwd_conds[t])
                def _fwd_wait(t=t):
                    f = pltpu.make_async_remote_copy(
                        src_ref=tile_ref(t), dst_ref=tile_ref(t),
                        send_sem=fwd_sems.at[t], recv_sem=recv_sems.at[t],
                        device_id=(2,), device_id_type=pl.DeviceIdType.MESH)
                    f.wait_send()

    out = pl.pallas_call(
        body,
        out_shape=jax.ShapeDtypeStruct((SQ, D), bf),
        in_specs=[pl.BlockSpec(memory_space=pltpu.VMEM)] * 5,
        out_specs=pl.BlockSpec(memory_space=pltpu.VMEM),
        scratch_shapes=[
            pltpu.VMEM((SQ, D), bf),
            pltpu.VMEM((QT, D), bf),
            pltpu.SemaphoreType.DMA((N_TILES, 2)),
            pltpu.SemaphoreType.DMA((N_TILES,)),
            pltpu.SemaphoreType.DMA((N_TILES,)),
        ],
        compiler_params=pltpu.CompilerParams(collective_id=0),
    )(x2, Wqb, K2, V2, Wob)

    return out.reshape(1, SQ, D)


# device time: 76403 ns/iter; 2.2149x vs baseline; 1.0104x over previous
import jax
import jax.numpy as jnp
from jax import lax
from jax.experimental import pallas as pl
from jax.experimental.pallas import tpu as pltpu

SQ = 2048
D = 1024
HQ = 8
DH = 128
BLK = 64
SCALE = 0.08838834764831843
TILES = [(0, 256), (256, 256), (512, 256), (768, 256), (1024, 256),
         (1280, 256), (1536, 256), (1792, 128), (1920, 128)]
N_TILES = len(TILES)
NT = (((1,), (1,)), ((), ()))
NN = (((1,), (0,)), ((), ()))


def kernel(x, Wq, K_ext, V_ext, Wo):
    bf = jnp.bfloat16
    f32 = jnp.float32
    x2 = x.reshape(SQ, D).astype(bf)
    K2 = K_ext.reshape(SQ, D).astype(bf)
    V2 = V_ext.reshape(SQ, D).astype(bf)
    Wqb = Wq.astype(bf)
    Wob = Wo.astype(bf)

    def body(x_ref, wq_ref, k_ref, v_ref, wo_ref, out_ref,
             qbf, vaug, ctx_tile, send_sems, recv_sems, fwd_sems):
        my = lax.axis_index("i")

        def tile_ref(t):
            r0, rows = TILES[t]
            return out_ref.at[r0:r0 + rows, :]

        barrier = pltpu.get_barrier_semaphore()

        @pl.when((my == 1) | (my == 3))
        def _():
            pl.semaphore_signal(barrier, inc=1, device_id=(0,),
                                device_id_type=pl.DeviceIdType.MESH)
            pl.semaphore_wait(barrier, 1)

        @pl.when(my == 2)
        def _():
            for d in (1, 3):
                pl.semaphore_signal(barrier, inc=1, device_id=(d,),
                                    device_id_type=pl.DeviceIdType.MESH)

        @pl.when(my == 0)
        def _():
            pl.semaphore_wait(barrier, 2)

        @pl.when(my == 0)
        def _producer():
            qbf[...] = (jnp.dot(x_ref[...], wq_ref[...],
                                preferred_element_type=f32)
                        * SCALE).astype(bf)
            ones = jnp.ones((SQ, DH), bf)
            for h in range(HQ):
                vaug[:, h * 2 * DH:h * 2 * DH + DH] = \
                    v_ref[:, h * DH:(h + 1) * DH]
                vaug[:, h * 2 * DH + DH:(h + 1) * 2 * DH] = ones
            biases = {}
            for rows in {256, 128}:
                r = lax.broadcasted_iota(jnp.int32, (rows, rows), 0) // BLK
                c = lax.broadcasted_iota(jnp.int32, (rows, rows), 1) // BLK
                biases[rows] = jnp.where(r >= c, f32(0), f32(-1e9)).astype(bf)

            rdmas = []
            for t in range(N_TILES):
                r0, rows = TILES[t]
                for h in range(HQ):
                    hs = slice(h * DH, (h + 1) * DH)
                    vs = slice(h * 2 * DH, (h + 1) * 2 * DH)
                    qt = qbf[r0:r0 + rows, hs]
                    sd = lax.dot_general(
                        qt, k_ref[r0:r0 + rows, hs], NT,
                        preferred_element_type=f32)
                    wd = jnp.exp(sd.astype(bf) + biases[rows])
                    res = lax.dot_general(
                        wd, vaug[r0:r0 + rows, vs], NN,
                        preferred_element_type=f32)
                    if r0 > 0:
                        s1 = lax.dot_general(
                            qt, k_ref[0:r0, hs], NT,
                            preferred_element_type=f32)
                        w1 = jnp.exp(s1.astype(bf))
                        res = res + lax.dot_general(
                            w1, vaug[0:r0, vs], NN,
                            preferred_element_type=f32)
                    ctx_tile[0:rows, hs] = \
                        (res[:, 0:DH] / res[:, DH:DH + 1]).astype(bf)
                out_ref[r0:r0 + rows, :] = jnp.dot(
                    ctx_tile[0:rows, :], wo_ref[...],
                    preferred_element_type=f32).astype(bf)
                for i, d in enumerate((1, 3)):
                    rd = pltpu.make_async_remote_copy(
                        src_ref=tile_ref(t), dst_ref=tile_ref(t),
                        send_sem=send_sems.at[t, i], recv_sem=recv_sems.at[t],
                        device_id=(d,), device_id_type=pl.DeviceIdType.MESH)
                    rd.start()
                    rdmas.append(rd)
            for rd in rdmas:
                rd.wait_send()

        @pl.when(my != 0)
        def _consumer():
            fwd_conds = []
            for t in range(N_TILES):
                rd = pltpu.make_async_remote_copy(
                    src_ref=tile_ref(t), dst_ref=tile_ref(t),
                    send_sem=send_sems.at[t, 0], recv_sem=recv_sems.at[t],
                    device_id=(0,), device_id_type=pl.DeviceIdType.MESH)
                rd.wait_recv()
                cond = (my == 1) if t % 2 == 0 else (my == 3)
                fwd_conds.append(cond)

                @pl.when(cond)
                def _fwd(t=t):
                    f = pltpu.make_async_remote_copy(
                        src_ref=tile_ref(t), dst_ref=tile_ref(t),
                        send_sem=fwd_sems.at[t], recv_sem=recv_sems.at[t],
                        device_id=(2,), device_id_type=pl.DeviceIdType.MESH)
                    f.start()

            for t in range(N_TILES):
                @pl.when(fwd_conds[t])
                def _fwd_wait(t=t):
                    f = pltpu.make_async_remote_copy(
                        src_ref=tile_ref(t), dst_ref=tile_ref(t),
                        send_sem=fwd_sems.at[t], recv_sem=recv_sems.at[t],
                        device_id=(2,), device_id_type=pl.DeviceIdType.MESH)
                    f.wait_send()

    out = pl.pallas_call(
        body,
        out_shape=jax.ShapeDtypeStruct((SQ, D), bf),
        in_specs=[pl.BlockSpec(memory_space=pltpu.VMEM)] * 5,
        out_specs=pl.BlockSpec(memory_space=pltpu.VMEM),
        scratch_shapes=[
            pltpu.VMEM((SQ, D), bf),
            pltpu.VMEM((SQ, 2 * D), bf),
            pltpu.VMEM((256, D), bf),
            pltpu.SemaphoreType.DMA((N_TILES, 2)),
            pltpu.SemaphoreType.DMA((N_TILES,)),
            pltpu.SemaphoreType.DMA((N_TILES,)),
        ],
        compiler_params=pltpu.CompilerParams(collective_id=0),
    )(x2, Wqb, K2, V2, Wob)

    return out.reshape(1, SQ, D)
